# Initial kernel scaffold; baseline (speedup 1.0000x reference)
import jax
import jax.numpy as jnp
from jax.experimental import pallas as pl


def kernel(xyz1, xyz2, points1):
    raise NotImplementedError("write your pallas kernel here")


# speedup vs baseline: 78.5633x; 78.5633x over previous
import functools

import jax
import jax.numpy as jnp
from jax import lax
from jax.experimental import pallas as pl
from jax.experimental.pallas import tpu as pltpu
from jax.experimental.pallas import tpu_sc as plsc

B, C, N, S, D = 8, 3, 8192, 2048, 256
ST = 128
K = 3


def _top3_body(x1_ref, x2t_ref, idx_ref, w_ref):
    b = pl.program_id(0)
    x1 = x1_ref[0]
    x2 = x2t_ref[0]

    mm = lax.dot_general(x2, x1, (((1,), (0,)), ((), ())),
                         preferred_element_type=jnp.float32)
    d = -2.0 * mm
    n2 = jnp.sum(x2 * x2, axis=1, keepdims=True)
    d = d + n2
    sq = x1 * x1
    n1 = (sq[0:1, :] + sq[1:2, :]) + sq[2:3, :]
    d = d + n1

    iota = lax.broadcasted_iota(jnp.int32, (ST, N), 1)
    inf = jnp.float32(jnp.inf)
    mins = []
    amins = []
    for k in range(K):
        m = jnp.min(d, axis=1, keepdims=True)
        a = jnp.min(jnp.where(d == m, iota, N), axis=1,
                    keepdims=True)
        mins.append(m)
        amins.append(a)
        if k < K - 1:
            d = jnp.where(iota == a, inf, d)

    r = [1.0 / (m + 1e-08) for m in mins]
    norm = r[0] + r[1] + r[2]
    w = [rk / norm for rk in r]

    lane = lax.broadcasted_iota(jnp.int32, (ST, 8), 1)
    boffs = b * N
    iv = jnp.broadcast_to(amins[0] + boffs, (ST, 8))
    iv = jnp.where(lane == 1, jnp.broadcast_to(amins[1] + boffs, (ST, 8)), iv)
    iv = jnp.where(lane == 2, jnp.broadcast_to(amins[2] + boffs, (ST, 8)), iv)
    iv = jnp.where(lane >= 3, 0, iv)
    idx_ref[0] = iv

    wv = jnp.broadcast_to(w[0], (ST, 8))
    wv = jnp.where(lane == 1, jnp.broadcast_to(w[1], (ST, 8)), wv)
    wv = jnp.where(lane == 2, jnp.broadcast_to(w[2], (ST, 8)), wv)
    wv = jnp.where(lane >= 3, jnp.float32(0.0), wv)
    w_ref[0] = wv


def _top3(xyz1, xyz2t):
    return pl.pallas_call(
        _top3_body,
        grid=(B, S // ST),
        in_specs=[
            pl.BlockSpec((1, C, N), lambda b, s: (b, 0, 0)),
            pl.BlockSpec((1, ST, C), lambda b, s: (b, s, 0)),
        ],
        out_specs=[
            pl.BlockSpec((1, ST, 8), lambda b, s: (b, s, 0)),
            pl.BlockSpec((1, ST, 8), lambda b, s: (b, s, 0)),
        ],
        out_shape=[
            jax.ShapeDtypeStruct((B, S, 8), jnp.int32),
            jax.ShapeDtypeStruct((B, S, 8), jnp.float32),
        ],
    )(xyz1, xyz2t)


BS = B * S
NW = 32
QPW = BS // NW
QC = 32
RPC = QC * K
NCH = QPW // QC


def _sc_combine_body(table_hbm, idx_hbm, w_hbm, out_hbm,
                     idx_v, w_v, rows_v, out_v, sem):
    wid = lax.axis_index("s") * 2 + lax.axis_index("c")

    def chunk(ci, carry):
        q0 = wid * QPW + ci * QC
        r0 = q0 * K
        pltpu.sync_copy(idx_hbm.at[pl.ds(r0, RPC)], idx_v)
        pltpu.sync_copy(w_hbm.at[pl.ds(r0, RPC)], w_v.at[pl.ds(0, RPC)])
        pltpu.async_copy(table_hbm.at[idx_v], rows_v, sem).wait()

        def qbody(q, c2):
            wvec = w_v[pl.ds(q * K, 16)]
            w0 = wvec[0]
            w1 = wvec[1]
            w2 = wvec[2]
            for dc in range(D // 16):
                sl = pl.ds(dc * 16, 16)
                acc = rows_v[q * K + 0, sl] * w0
                acc = acc + rows_v[q * K + 1, sl] * w1
                acc = acc + rows_v[q * K + 2, sl] * w2
                out_v[q, sl] = acc
            return c2

        lax.fori_loop(0, QC, qbody, 0)
        pltpu.sync_copy(out_v, out_hbm.at[pl.ds(q0, QC)])
        return carry

    lax.fori_loop(0, NCH, chunk, 0)


@functools.partial(jax.jit, static_argnames=())
def _sc_combine(table, idx, w):
    mesh = plsc.VectorSubcoreMesh(core_axis_name="c", subcore_axis_name="s")
    fn = functools.partial(
        pl.kernel,
        out_type=jax.ShapeDtypeStruct((BS, D), jnp.float32),
        mesh=mesh,
        scratch_types=[
            pltpu.VMEM((RPC,), jnp.int32),
            pltpu.VMEM((RPC + 16,), jnp.float32),
            pltpu.VMEM((RPC, D), jnp.float32),
            pltpu.VMEM((QC, D), jnp.float32),
            pltpu.SemaphoreType.DMA,
        ],
    )(_sc_combine_body)
    return fn(table, idx, w)


def kernel(xyz1, xyz2, points1):
    idx8, w8 = _top3(xyz1, jnp.transpose(xyz2, (0, 2, 1)))
    idx3 = idx8[:, :, :K].reshape(-1)
    w3 = w8[:, :, :K].reshape(-1)
    table = jnp.transpose(points1, (0, 2, 1)).reshape(B * N, D)
    out_rows = _sc_combine(table, idx3, w3)
    return jnp.transpose(out_rows.reshape(B, S, D), (0, 2, 1))


# speedup vs baseline: 89.9253x; 1.1446x over previous
import functools

import jax
import jax.numpy as jnp
from jax import lax
from jax.experimental import pallas as pl
from jax.experimental.pallas import tpu as pltpu
from jax.experimental.pallas import tpu_sc as plsc

B, C, N, S, D = 8, 3, 8192, 2048, 256
ST = 128
K = 3


def _top3_body(x1_ref, x2t_ref, idx_ref, w_ref):
    b = pl.program_id(0)
    x1 = x1_ref[0]
    x2 = x2t_ref[0]

    mm = lax.dot_general(x2, x1, (((1,), (0,)), ((), ())),
                         preferred_element_type=jnp.float32)
    d = -2.0 * mm
    n2 = jnp.sum(x2 * x2, axis=1, keepdims=True)
    d = d + n2
    sq = x1 * x1
    n1 = (sq[0:1, :] + sq[1:2, :]) + sq[2:3, :]
    d = d + n1

    iota = lax.broadcasted_iota(jnp.int32, (ST, N), 1).astype(jnp.float32)
    inf = jnp.float32(jnp.inf)
    bigf = jnp.float32(N)
    mins = []
    amins = []
    for k in range(K):
        m = jnp.min(d, axis=1, keepdims=True)
        af = jnp.min(jnp.where(d == m, iota, bigf), axis=1,
                     keepdims=True)
        mins.append(m)
        amins.append(af.astype(jnp.int32))
        if k < K - 1:
            d = jnp.where(iota == af, inf, d)

    r = [1.0 / (m + 1e-08) for m in mins]
    norm = r[0] + r[1] + r[2]
    w = [rk / norm for rk in r]

    lane = lax.broadcasted_iota(jnp.int32, (ST, 8), 1)
    boffs = b * N
    iv = jnp.broadcast_to(amins[0] + boffs, (ST, 8))
    iv = jnp.where(lane == 1, jnp.broadcast_to(amins[1] + boffs, (ST, 8)), iv)
    iv = jnp.where(lane == 2, jnp.broadcast_to(amins[2] + boffs, (ST, 8)), iv)
    iv = jnp.where(lane >= 3, 0, iv)
    idx_ref[0] = iv

    wv = jnp.broadcast_to(w[0], (ST, 8))
    wv = jnp.where(lane == 1, jnp.broadcast_to(w[1], (ST, 8)), wv)
    wv = jnp.where(lane == 2, jnp.broadcast_to(w[2], (ST, 8)), wv)
    wv = jnp.where(lane >= 3, jnp.float32(0.0), wv)
    w_ref[0] = wv


def _top3(xyz1, xyz2t):
    return pl.pallas_call(
        _top3_body,
        grid=(B, S // ST),
        in_specs=[
            pl.BlockSpec((1, C, N), lambda b, s: (b, 0, 0)),
            pl.BlockSpec((1, ST, C), lambda b, s: (b, s, 0)),
        ],
        out_specs=[
            pl.BlockSpec((1, ST, 8), lambda b, s: (b, s, 0)),
            pl.BlockSpec((1, ST, 8), lambda b, s: (b, s, 0)),
        ],
        out_shape=[
            jax.ShapeDtypeStruct((B, S, 8), jnp.int32),
            jax.ShapeDtypeStruct((B, S, 8), jnp.float32),
        ],
    )(xyz1, xyz2t)


BS = B * S
NW = 32
QPW = BS // NW
QC = 32
RPC = QC * K
NCH = QPW // QC


def _sc_combine_body(table_hbm, idx_hbm, w_hbm, out_hbm,
                     idx_v, w_v, rows_v, out_v, sem):
    wid = lax.axis_index("s") * 2 + lax.axis_index("c")

    def chunk(ci, carry):
        q0 = wid * QPW + ci * QC
        r0 = q0 * K
        pltpu.sync_copy(idx_hbm.at[pl.ds(r0, RPC)], idx_v)
        pltpu.sync_copy(w_hbm.at[pl.ds(r0, RPC)], w_v.at[pl.ds(0, RPC)])
        pltpu.async_copy(table_hbm.at[idx_v], rows_v, sem).wait()

        def qbody(q, c2):
            wvec = w_v[pl.ds(q * K, 16)]
            w0 = wvec[0]
            w1 = wvec[1]
            w2 = wvec[2]
            for dc in range(D // 16):
                sl = pl.ds(dc * 16, 16)
                acc = rows_v[q * K + 0, sl] * w0
                acc = acc + rows_v[q * K + 1, sl] * w1
                acc = acc + rows_v[q * K + 2, sl] * w2
                out_v[q, sl] = acc
            return c2

        lax.fori_loop(0, QC, qbody, 0)
        pltpu.sync_copy(out_v, out_hbm.at[pl.ds(q0, QC)])
        return carry

    lax.fori_loop(0, NCH, chunk, 0)


@functools.partial(jax.jit, static_argnames=())
def _sc_combine(table, idx, w):
    mesh = plsc.VectorSubcoreMesh(core_axis_name="c", subcore_axis_name="s")
    fn = functools.partial(
        pl.kernel,
        out_type=jax.ShapeDtypeStruct((BS, D), jnp.float32),
        mesh=mesh,
        scratch_types=[
            pltpu.VMEM((RPC,), jnp.int32),
            pltpu.VMEM((RPC + 16,), jnp.float32),
            pltpu.VMEM((RPC, D), jnp.float32),
            pltpu.VMEM((QC, D), jnp.float32),
            pltpu.SemaphoreType.DMA,
        ],
    )(_sc_combine_body)
    return fn(table, idx, w)


def kernel(xyz1, xyz2, points1):
    idx8, w8 = _top3(xyz1, jnp.transpose(xyz2, (0, 2, 1)))
    idx3 = idx8[:, :, :K].reshape(-1)
    w3 = w8[:, :, :K].reshape(-1)
    table = jnp.transpose(points1, (0, 2, 1)).reshape(B * N, D)
    out_rows = _sc_combine(table, idx3, w3)
    return jnp.transpose(out_rows.reshape(B, S, D), (0, 2, 1))


# speedup vs baseline: 96.1793x; 1.0695x over previous
import functools

import jax
import jax.numpy as jnp
from jax import lax
from jax.experimental import pallas as pl
from jax.experimental.pallas import tpu as pltpu
from jax.experimental.pallas import tpu_sc as plsc

B, C, N, S, D = 8, 3, 8192, 2048, 256
ST = 256
K = 3


def _top3_body(x1_ref, x2t_ref, idx_ref, w_ref):
    b = pl.program_id(0)
    x1 = x1_ref[0]
    x2 = x2t_ref[0]

    mm = lax.dot_general(x2, x1, (((1,), (0,)), ((), ())),
                         preferred_element_type=jnp.float32)
    d = -2.0 * mm
    n2 = jnp.sum(x2 * x2, axis=1, keepdims=True)
    d = d + n2
    sq = x1 * x1
    n1 = (sq[0:1, :] + sq[1:2, :]) + sq[2:3, :]
    d = d + n1

    iota = lax.broadcasted_iota(jnp.int32, (ST, N), 1).astype(jnp.float32)
    inf = jnp.float32(jnp.inf)
    bigf = jnp.float32(N)
    mins = []
    amins = []
    for k in range(K):
        m = jnp.min(d, axis=1, keepdims=True)
        af = jnp.min(jnp.where(d == m, iota, bigf), axis=1,
                     keepdims=True)
        mins.append(m)
        amins.append(af.astype(jnp.int32))
        if k < K - 1:
            d = jnp.where(iota == af, inf, d)

    r = [1.0 / (m + 1e-08) for m in mins]
    norm = r[0] + r[1] + r[2]
    w = [rk / norm for rk in r]

    lane = lax.broadcasted_iota(jnp.int32, (ST, 8), 1)
    boffs = b * N
    iv = jnp.broadcast_to(amins[0] + boffs, (ST, 8))
    iv = jnp.where(lane == 1, jnp.broadcast_to(amins[1] + boffs, (ST, 8)), iv)
    iv = jnp.where(lane == 2, jnp.broadcast_to(amins[2] + boffs, (ST, 8)), iv)
    iv = jnp.where(lane >= 3, 0, iv)
    idx_ref[0] = iv

    wv = jnp.broadcast_to(w[0], (ST, 8))
    wv = jnp.where(lane == 1, jnp.broadcast_to(w[1], (ST, 8)), wv)
    wv = jnp.where(lane == 2, jnp.broadcast_to(w[2], (ST, 8)), wv)
    wv = jnp.where(lane >= 3, jnp.float32(0.0), wv)
    w_ref[0] = wv


def _top3(xyz1, xyz2t):
    return pl.pallas_call(
        _top3_body,
        grid=(B, S // ST),
        in_specs=[
            pl.BlockSpec((1, C, N), lambda b, s: (b, 0, 0)),
            pl.BlockSpec((1, ST, C), lambda b, s: (b, s, 0)),
        ],
        out_specs=[
            pl.BlockSpec((1, ST, 8), lambda b, s: (b, s, 0)),
            pl.BlockSpec((1, ST, 8), lambda b, s: (b, s, 0)),
        ],
        out_shape=[
            jax.ShapeDtypeStruct((B, S, 8), jnp.int32),
            jax.ShapeDtypeStruct((B, S, 8), jnp.float32),
        ],
    )(xyz1, xyz2t)


BS = B * S
NW = 32
QPW = BS // NW
QC = 32
RPC = QC * K
NCH = QPW // QC


def _sc_combine_body(table_hbm, idx_hbm, w_hbm, out_hbm,
                     idx_v, w_v, rows_v, out_v, sem):
    wid = lax.axis_index("s") * 2 + lax.axis_index("c")

    def chunk(ci, carry):
        q0 = wid * QPW + ci * QC
        r0 = q0 * K
        pltpu.sync_copy(idx_hbm.at[pl.ds(r0, RPC)], idx_v)
        pltpu.sync_copy(w_hbm.at[pl.ds(r0, RPC)], w_v.at[pl.ds(0, RPC)])
        pltpu.async_copy(table_hbm.at[idx_v], rows_v, sem).wait()

        def qbody(q, c2):
            wvec = w_v[pl.ds(q * K, 16)]
            w0 = wvec[0]
            w1 = wvec[1]
            w2 = wvec[2]
            for dc in range(D // 16):
                sl = pl.ds(dc * 16, 16)
                acc = rows_v[q * K + 0, sl] * w0
                acc = acc + rows_v[q * K + 1, sl] * w1
                acc = acc + rows_v[q * K + 2, sl] * w2
                out_v[q, sl] = acc
            return c2

        lax.fori_loop(0, QC, qbody, 0)
        pltpu.sync_copy(out_v, out_hbm.at[pl.ds(q0, QC)])
        return carry

    lax.fori_loop(0, NCH, chunk, 0)


@functools.partial(jax.jit, static_argnames=())
def _sc_combine(table, idx, w):
    mesh = plsc.VectorSubcoreMesh(core_axis_name="c", subcore_axis_name="s")
    fn = functools.partial(
        pl.kernel,
        out_type=jax.ShapeDtypeStruct((BS, D), jnp.float32),
        mesh=mesh,
        scratch_types=[
            pltpu.VMEM((RPC,), jnp.int32),
            pltpu.VMEM((RPC + 16,), jnp.float32),
            pltpu.VMEM((RPC, D), jnp.float32),
            pltpu.VMEM((QC, D), jnp.float32),
            pltpu.SemaphoreType.DMA,
        ],
    )(_sc_combine_body)
    return fn(table, idx, w)


def kernel(xyz1, xyz2, points1):
    idx8, w8 = _top3(xyz1, jnp.transpose(xyz2, (0, 2, 1)))
    idx3 = idx8[:, :, :K].reshape(-1)
    w3 = w8[:, :, :K].reshape(-1)
    table = jnp.transpose(points1, (0, 2, 1)).reshape(B * N, D)
    out_rows = _sc_combine(table, idx3, w3)
    return jnp.transpose(out_rows.reshape(B, S, D), (0, 2, 1))


# speedup vs baseline: 103.3069x; 1.0741x over previous
import functools

import jax
import jax.numpy as jnp
from jax import lax
from jax.experimental import pallas as pl
from jax.experimental.pallas import tpu as pltpu
from jax.experimental.pallas import tpu_sc as plsc

B, C, N, S, D = 8, 3, 8192, 2048, 256
ST = 256
K = 3


def _top3_body(x1_ref, x2t_ref, idx_ref, w_ref):
    b = pl.program_id(0)
    x1 = x1_ref[0]
    x2 = x2t_ref[0]

    mm = lax.dot_general(x2, x1, (((1,), (0,)), ((), ())),
                         preferred_element_type=jnp.float32)
    d = -2.0 * mm
    n2 = jnp.sum(x2 * x2, axis=1, keepdims=True)
    d = d + n2
    sq = x1 * x1
    n1 = (sq[0:1, :] + sq[1:2, :]) + sq[2:3, :]
    d = d + n1

    iota = lax.broadcasted_iota(jnp.int32, (ST, N), 1).astype(jnp.float32)
    inf = jnp.float32(jnp.inf)
    bigf = jnp.float32(N)
    mins = []
    amins = []
    for k in range(K):
        m = jnp.min(d, axis=1, keepdims=True)
        af = jnp.min(jnp.where(d == m, iota, bigf), axis=1,
                     keepdims=True)
        mins.append(m)
        amins.append(af.astype(jnp.int32))
        if k < K - 1:
            d = jnp.where(iota == af, inf, d)

    r = [1.0 / (m + 1e-08) for m in mins]
    norm = r[0] + r[1] + r[2]
    w = [rk / norm for rk in r]

    lane = lax.broadcasted_iota(jnp.int32, (ST, 8), 1)
    boffs = b * N
    iv = jnp.broadcast_to(amins[0] + boffs, (ST, 8))
    iv = jnp.where(lane == 1, jnp.broadcast_to(amins[1] + boffs, (ST, 8)), iv)
    iv = jnp.where(lane == 2, jnp.broadcast_to(amins[2] + boffs, (ST, 8)), iv)
    iv = jnp.where(lane >= 3, 0, iv)
    idx_ref[0] = iv

    wv = jnp.broadcast_to(w[0], (ST, 8))
    wv = jnp.where(lane == 1, jnp.broadcast_to(w[1], (ST, 8)), wv)
    wv = jnp.where(lane == 2, jnp.broadcast_to(w[2], (ST, 8)), wv)
    wv = jnp.where(lane >= 3, jnp.float32(0.0), wv)
    w_ref[0] = wv


def _top3(xyz1, xyz2t):
    return pl.pallas_call(
        _top3_body,
        grid=(B, S // ST),
        in_specs=[
            pl.BlockSpec((1, C, N), lambda b, s: (b, 0, 0)),
            pl.BlockSpec((1, ST, C), lambda b, s: (b, s, 0)),
        ],
        out_specs=[
            pl.BlockSpec((1, ST, 8), lambda b, s: (b, s, 0)),
            pl.BlockSpec((1, ST, 8), lambda b, s: (b, s, 0)),
        ],
        out_shape=[
            jax.ShapeDtypeStruct((B, S, 8), jnp.int32),
            jax.ShapeDtypeStruct((B, S, 8), jnp.float32),
        ],
    )(xyz1, xyz2t)


BS = B * S
NW = 32
QPW = BS // NW
QC = 32
RPC = QC * K
NCH = QPW // QC


def _sc_combine_body(table_hbm, idx_hbm, w_hbm, out_hbm,
                     idx_v, w_v, rows_v, out_v, sem0, sem1):
    wid = lax.axis_index("s") * 2 + lax.axis_index("c")
    q_base = wid * QPW
    r_base = q_base * K
    sems = (sem0, sem1)

    pltpu.sync_copy(idx_hbm.at[pl.ds(r_base, QPW * K)], idx_v)
    pltpu.sync_copy(w_hbm.at[pl.ds(r_base, QPW * K)],
                    w_v.at[pl.ds(0, QPW * K)])
    pltpu.async_copy(table_hbm.at[idx_v.at[pl.ds(0, RPC)]],
                     rows_v.at[0], sem0)

    def pair(ch, carry):
        for sub in range(2):
            ci = ch * 2 + sub
            p = sub

            @pl.when(ci + 1 < NCH)
            def _():
                pltpu.async_copy(
                    table_hbm.at[idx_v.at[pl.ds((ci + 1) * RPC, RPC)]],
                    rows_v.at[1 - p], sems[1 - p])

            pltpu.make_async_copy(table_hbm.at[pl.ds(0, RPC)],
                                  rows_v.at[p], sems[p]).wait()

            def qbody(q, c2):
                wvec = w_v[pl.ds(ci * RPC + q * K, 16)]
                w0 = wvec[0]
                w1 = wvec[1]
                w2 = wvec[2]
                for dc in range(D // 16):
                    sl = pl.ds(dc * 16, 16)
                    acc = rows_v[p, q * K + 0, sl] * w0
                    acc = acc + rows_v[p, q * K + 1, sl] * w1
                    acc = acc + rows_v[p, q * K + 2, sl] * w2
                    out_v[q, sl] = acc
                return c2

            lax.fori_loop(0, QC, qbody, 0)
            pltpu.sync_copy(out_v, out_hbm.at[pl.ds(q_base + ci * QC, QC)])
        return carry

    lax.fori_loop(0, NCH // 2, pair, 0)


@functools.partial(jax.jit, static_argnames=())
def _sc_combine(table, idx, w):
    mesh = plsc.VectorSubcoreMesh(core_axis_name="c", subcore_axis_name="s")
    fn = functools.partial(
        pl.kernel,
        out_type=jax.ShapeDtypeStruct((BS, D), jnp.float32),
        mesh=mesh,
        scratch_types=[
            pltpu.VMEM((QPW * K,), jnp.int32),
            pltpu.VMEM((QPW * K + 16,), jnp.float32),
            pltpu.VMEM((2, RPC, D), jnp.float32),
            pltpu.VMEM((QC, D), jnp.float32),
            pltpu.SemaphoreType.DMA,
            pltpu.SemaphoreType.DMA,
        ],
    )(_sc_combine_body)
    return fn(table, idx, w)


def kernel(xyz1, xyz2, points1):
    idx8, w8 = _top3(xyz1, jnp.transpose(xyz2, (0, 2, 1)))
    idx3 = idx8[:, :, :K].reshape(-1)
    w3 = w8[:, :, :K].reshape(-1)
    table = jnp.transpose(points1, (0, 2, 1)).reshape(B * N, D)
    out_rows = _sc_combine(table, idx3, w3)
    return jnp.transpose(out_rows.reshape(B, S, D), (0, 2, 1))
